# baseline (speedup 1.0000x reference)
import dataclasses
import functools

import jax
import jax.numpy as jnp
from jax import lax
from jax.experimental import pallas as pl
from jax.experimental.pallas import tpu as pltpu
from jax.experimental.pallas import tpu_sc as plsc

BATCH = 16384
HIST = 200
DPAD = 8
NW = 32
PER_W = BATCH // NW
G = 16
NGROUPS = PER_W // G
ROWS = G * HIST

_LOG2E = 1.4426950408889634
_LN2_HI = 0.693145751953125
_LN2_LO = 1.4286067653302226e-06


def _exp_f32(x):
    t = x * _LOG2E
    ni = (t + jnp.where(t >= 0.0, 0.5, -0.5)).astype(jnp.int32)
    ni = jnp.clip(ni, -126, 127)
    nf = ni.astype(jnp.float32)
    r = x - nf * _LN2_HI
    r = r - nf * _LN2_LO
    p = 1.0 / 720.0
    for c in (1.0 / 120.0, 1.0 / 24.0, 1.0 / 6.0, 0.5, 1.0, 1.0):
        p = p * r + c
    scale = lax.bitcast_convert_type((ni + 127) << 23, jnp.float32)
    return p * scale


def _sc_body(w_hbm, lflat_hbm, x_hbm, y_hbm, out_hbm,
             xv, exall, yv, lbuf, ey, outb, sem):
    cid = lax.axis_index("c")
    sid = lax.axis_index("s")
    wid = sid * 2 + cid
    wbase = wid * PER_W

    pltpu.sync_copy(x_hbm.at[pl.ds(wbase, PER_W)], xv)
    pltpu.sync_copy(y_hbm.at[pl.ds(wbase, PER_W)], yv)
    pltpu.async_copy(w_hbm.at[xv], exall, sem).wait()

    iota = lax.iota(jnp.int32, G)
    rowbase = iota * HIST

    @pl.loop(0, NGROUPS)
    def _group(g):
        pltpu.sync_copy(lflat_hbm.at[pl.ds((wbase + g * G) * HIST, ROWS)], lbuf)
        pltpu.async_copy(w_hbm.at[lbuf], ey, sem).wait()

        gxrow = iota + g * G
        exd = [plsc.load_gather(exall, [gxrow, jnp.full((G,), d, jnp.int32)])
               for d in range(5)]
        ygrp = yv[pl.ds(g * G, G)]

        def step(k, carry):
            denom, numer = carry
            rowv = rowbase + k
            sc = exd[0] * plsc.load_gather(ey, [rowv, jnp.full((G,), 0, jnp.int32)])
            for d in range(1, 5):
                sc = sc + exd[d] * plsc.load_gather(
                    ey, [rowv, jnp.full((G,), d, jnp.int32)])
            lvals = plsc.load_gather(lbuf, [rowv])
            masked = jnp.where(lvals > 0, _exp_f32(sc), 0.0)
            denom = denom + masked
            numer = jnp.where(ygrp == k, masked, numer)
            return denom, numer

        zeros = jnp.zeros((G,), jnp.float32)
        denom, numer = lax.fori_loop(0, HIST, step, (zeros, zeros))
        outb[pl.ds(g * G, G)] = numer / denom

    pltpu.sync_copy(outb, out_hbm.at[pl.ds(wbase, PER_W)])


def kernel(x, y, l, W):
    w8 = jnp.pad(W, ((0, 0), (0, DPAD - W.shape[1])))
    lflat = l.reshape(-1).astype(jnp.int32)
    mesh = plsc.VectorSubcoreMesh(core_axis_name="c", subcore_axis_name="s")
    cp = pltpu.CompilerParams()
    for fld, val in (("needs_layout_passes", False),
                     ("use_tc_tiling_on_sc", False)):
        if fld in pltpu.CompilerParams.__dataclass_fields__:
            cp = dataclasses.replace(cp, **{fld: val})
    run = pl.kernel(
        _sc_body,
        out_type=jax.ShapeDtypeStruct((BATCH,), jnp.float32),
        mesh=mesh,
        scratch_types=[
            pltpu.VMEM((PER_W,), jnp.int32),
            pltpu.VMEM((PER_W, DPAD), jnp.float32),
            pltpu.VMEM((PER_W,), jnp.int32),
            pltpu.VMEM((ROWS,), jnp.int32),
            pltpu.VMEM((ROWS, DPAD), jnp.float32),
            pltpu.VMEM((PER_W,), jnp.float32),
            pltpu.SemaphoreType.DMA,
        ],
        compiler_params=cp,
    )
    return run(w8, lflat, x.astype(jnp.int32), y.astype(jnp.int32))


# speedup vs baseline: 1.0324x; 1.0324x over previous
import dataclasses
import functools

import jax
import jax.numpy as jnp
from jax import lax
from jax.experimental import pallas as pl
from jax.experimental.pallas import tpu as pltpu
from jax.experimental.pallas import tpu_sc as plsc

BATCH = 16384
HIST = 200
DPAD = 8
NW = 32
PER_W = BATCH // NW
G = 16
NGROUPS = PER_W // G
ROWS = G * HIST

_LOG2E = 1.4426950408889634
_LN2_HI = 0.693145751953125
_LN2_LO = 1.4286067653302226e-06


def _exp_f32(x):
    t = x * _LOG2E
    ni = (t + jnp.where(t >= 0.0, 0.5, -0.5)).astype(jnp.int32)
    ni = jnp.clip(ni, -126, 127)
    nf = ni.astype(jnp.float32)
    r = x - nf * _LN2_HI
    r = r - nf * _LN2_LO
    p = 1.0 / 720.0
    for c in (1.0 / 120.0, 1.0 / 24.0, 1.0 / 6.0, 0.5, 1.0, 1.0):
        p = p * r + c
    scale = lax.bitcast_convert_type((ni + 127) << 23, jnp.float32)
    return p * scale


def _sc_body(w_hbm, lflat_hbm, x_hbm, y_hbm, out_hbm,
             xv, exall, yv, lbuf, ey, outb, sem):
    cid = lax.axis_index("c")
    sid = lax.axis_index("s")
    wid = sid * 2 + cid
    wbase = wid * PER_W

    pltpu.sync_copy(x_hbm.at[pl.ds(wbase, PER_W)], xv)
    pltpu.sync_copy(y_hbm.at[pl.ds(wbase, PER_W)], yv)
    pltpu.async_copy(w_hbm.at[xv], exall, sem).wait()

    iota = lax.iota(jnp.int32, G)
    rowbase = iota * HIST

    @pl.loop(0, NGROUPS)
    def _group(g):
        pltpu.sync_copy(lflat_hbm.at[pl.ds((wbase + g * G) * HIST, ROWS)], lbuf)
        pltpu.async_copy(w_hbm.at[lbuf], ey, sem).wait()

        gxrow = iota + g * G
        exd = [plsc.load_gather(exall, [gxrow, jnp.full((G,), d, jnp.int32)])
               for d in range(5)]
        ygrp = yv[pl.ds(g * G, G)]

        def step(k, carry):
            denom, numer = carry
            rowv = rowbase + k
            sc = exd[0] * plsc.load_gather(ey, [rowv, jnp.full((G,), 0, jnp.int32)])
            for d in range(1, 5):
                sc = sc + exd[d] * plsc.load_gather(
                    ey, [rowv, jnp.full((G,), d, jnp.int32)])
            lvals = plsc.load_gather(lbuf, [rowv])
            masked = jnp.where(lvals > 0, _exp_f32(sc), 0.0)
            denom = denom + masked
            numer = jnp.where(ygrp == k, masked, numer)
            return denom, numer

        zeros = jnp.zeros((G,), jnp.float32)
        denom, numer = lax.fori_loop(0, HIST, step, (zeros, zeros))
        outb[pl.ds(g * G, G)] = numer / denom

    pltpu.sync_copy(outb, out_hbm.at[pl.ds(wbase, PER_W)])


def kernel(x, y, l, W):
    w8 = jnp.pad(W, ((0, 0), (0, DPAD - W.shape[1])))
    w8 = w8.astype(jnp.bfloat16).astype(jnp.float32)
    lflat = l.reshape(-1).astype(jnp.int32)
    mesh = plsc.VectorSubcoreMesh(core_axis_name="c", subcore_axis_name="s")
    cp = pltpu.CompilerParams()
    for fld, val in (("needs_layout_passes", False),
                     ("use_tc_tiling_on_sc", False)):
        if fld in pltpu.CompilerParams.__dataclass_fields__:
            cp = dataclasses.replace(cp, **{fld: val})
    run = pl.kernel(
        _sc_body,
        out_type=jax.ShapeDtypeStruct((BATCH,), jnp.float32),
        mesh=mesh,
        scratch_types=[
            pltpu.VMEM((PER_W,), jnp.int32),
            pltpu.VMEM((PER_W, DPAD), jnp.float32),
            pltpu.VMEM((PER_W,), jnp.int32),
            pltpu.VMEM((ROWS,), jnp.int32),
            pltpu.VMEM((ROWS, DPAD), jnp.float32),
            pltpu.VMEM((PER_W,), jnp.float32),
            pltpu.SemaphoreType.DMA,
        ],
        compiler_params=cp,
    )
    return run(w8, lflat, x.astype(jnp.int32), y.astype(jnp.int32))
